# baseline (speedup 1.0000x reference)
import functools

import jax
import jax.numpy as jnp
from jax.experimental import pallas as pl
from jax.experimental.pallas import tpu as pltpu

_INTERPRET = False

E = 8
D = 768
H = 768
T = 4096
BTR = 512
BT = 128
NT = T // BT
NW = NT + E


def _router_body(x_ref, wg_ref, eid_ref, counts_ref, loss_ref, acc_ref):
    i = pl.program_id(0)
    n = pl.num_programs(0)
    logits = jnp.dot(x_ref[...], wg_ref[...], preferred_element_type=jnp.float32)
    lane = jax.lax.broadcasted_iota(jnp.int32, logits.shape, 1)
    logits = jnp.where(lane < E, logits, -jnp.inf)
    m = jnp.max(logits, axis=1, keepdims=True)
    eid = jnp.min(jnp.where(logits == m, lane, jnp.int32(2**30)), axis=1,
                  keepdims=True)
    eid_ref[...] = eid
    onehot = jnp.where((lane == eid) & (lane < E), jnp.float32(1.0),
                       jnp.float32(0.0))

    @pl.when(i == 0)
    def _():
        acc_ref[...] = jnp.zeros_like(acc_ref)

    acc_ref[...] += jnp.sum(onehot, axis=0, keepdims=True)

    @pl.when(i == n - 1)
    def _():
        c = acc_ref[...]
        counts_ref[...] = c
        lane1 = lane[:1, :]
        mean = jnp.sum(jnp.where(lane1 < E, c, 0.0)) / jnp.float32(E)
        dev = jnp.where(lane1 < E, c - mean, 0.0)
        var = jnp.sum(dev * dev) / jnp.float32(E - 1)
        loss_ref[...] = jnp.full((1, 1), var / (mean * mean + 1e-10),
                                 jnp.float32)


def _router(x, wg_pad):
    return pl.pallas_call(
        _router_body,
        grid=(T // BTR,),
        in_specs=[
            pl.BlockSpec((BTR, D), lambda i: (i, 0)),
            pl.BlockSpec((D, 128), lambda i: (0, 0)),
        ],
        out_specs=[
            pl.BlockSpec((BTR, 1), lambda i: (i, 0)),
            pl.BlockSpec((1, 128), lambda i: (0, 0)),
            pl.BlockSpec((1, 1), lambda i: (0, 0)),
        ],
        out_shape=[
            jax.ShapeDtypeStruct((T, 1), jnp.int32),
            jax.ShapeDtypeStruct((1, 128), jnp.float32),
            jax.ShapeDtypeStruct((1, 1), jnp.float32),
        ],
        scratch_shapes=[pltpu.VMEM((1, 128), jnp.float32)],
        interpret=_INTERPRET,
    )(x, wg_pad)


def _gmm_body(tile_ref, eidw_ref, offs_ref,
              x_ref, w1_ref, b1_ref, w2_ref, b2_ref, y_ref):
    w = pl.program_id(0)
    t = tile_ref[w]
    e = eidw_ref[w]
    s = offs_ref[e]
    epos = offs_ref[e + 1]
    rows = t * BT + jax.lax.broadcasted_iota(jnp.int32, (BT, 1), 0)
    mask = (rows >= s) & (rows < epos)
    xb = x_ref[...].astype(jnp.bfloat16)
    h = jnp.maximum(
        jnp.dot(xb, w1_ref[0].astype(jnp.bfloat16),
                preferred_element_type=jnp.float32) + b1_ref[0], 0.0)
    y = jnp.dot(h.astype(jnp.bfloat16), w2_ref[0].astype(jnp.bfloat16),
                preferred_element_type=jnp.float32) + b2_ref[0]
    y_ref[...] = jnp.where(mask, y, y_ref[...])


def _gmm(wu_tile, wu_eid, offs, x_sorted, W1, b1, W2, b2):
    grid_spec = pltpu.PrefetchScalarGridSpec(
        num_scalar_prefetch=3,
        grid=(NW,),
        in_specs=[
            pl.BlockSpec((BT, D), lambda w, tr, er, ofr: (tr[w], 0)),
            pl.BlockSpec((1, D, H), lambda w, tr, er, ofr: (er[w], 0, 0)),
            pl.BlockSpec((1, 1, H), lambda w, tr, er, ofr: (er[w], 0, 0)),
            pl.BlockSpec((1, H, D), lambda w, tr, er, ofr: (er[w], 0, 0)),
            pl.BlockSpec((1, 1, D), lambda w, tr, er, ofr: (er[w], 0, 0)),
        ],
        out_specs=pl.BlockSpec((BT, D), lambda w, tr, er, ofr: (tr[w], 0)),
    )
    return pl.pallas_call(
        _gmm_body,
        grid_spec=grid_spec,
        out_shape=jax.ShapeDtypeStruct((T, D), jnp.float32),
        compiler_params=pltpu.CompilerParams(
            dimension_semantics=("arbitrary",)),
        interpret=_INTERPRET,
    )(wu_tile, wu_eid, offs, x_sorted, W1,
      b1.reshape(E, 1, H), W2, b2.reshape(E, 1, D))


def _plan_work_units(offs9):
    interior = offs9[1:E]
    tstart = jnp.arange(NT, dtype=jnp.int32) * BT
    e_start = jnp.sum(interior[None, :] <= tstart[:, None], axis=1)
    e_end = jnp.sum(interior[None, :] <= (tstart + BT - 1)[:, None], axis=1)
    nw = e_end - e_start + 1
    starts = jnp.concatenate(
        [jnp.zeros((1,), jnp.int32), jnp.cumsum(nw)]).astype(jnp.int32)
    n = starts[NT]
    w = jnp.arange(NW, dtype=jnp.int32)
    wc = jnp.minimum(w, n - 1)
    t = jnp.sum(starts[None, :NT] <= wc[:, None], axis=1).astype(jnp.int32) - 1
    eid = e_start[t] + (wc - starts[t])
    return t.astype(jnp.int32), eid.astype(jnp.int32)


def kernel(x, Wg, W1, b1, W2, b2):
    wg_pad = jnp.zeros((D, 128), jnp.float32).at[:, :E].set(Wg)
    eid2d, counts128, loss11 = _router(x, wg_pad)
    eid = eid2d[:, 0]
    counts = counts128[0, :E].astype(jnp.int32)
    loss = loss11[0, 0]
    offs9 = jnp.concatenate(
        [jnp.zeros((1,), jnp.int32), jnp.cumsum(counts)]).astype(jnp.int32)
    offs16 = jnp.zeros((16,), jnp.int32).at[:E + 1].set(offs9)

    sort_idx = jnp.argsort(eid, stable=True)
    x_sorted = jnp.take(x, sort_idx, axis=0)

    wu_tile, wu_eid = _plan_work_units(offs9)
    y_sorted = _gmm(wu_tile, wu_eid, offs16, x_sorted, W1, b1, W2, b2)

    out = jnp.zeros((T, D), jnp.float32).at[sort_idx].set(y_sorted)
    return out, loss, loss


# speedup vs baseline: 1.1849x; 1.1849x over previous
import functools

import jax
import jax.numpy as jnp
from jax import lax
from jax.experimental import pallas as pl
from jax.experimental.pallas import tpu as pltpu
from jax.experimental.pallas import tpu_sc as plsc

_INTERPRET = False

E = 8
D = 768
H = 768
T = 4096
BTR = 512
BT = 128
NT = T // BT
NW = NT + E

NWORK = 32
TPW = T // NWORK


def _router_body(x_ref, wg_ref, eid_ref, rnk_ref, counts_ref, blk_ref,
                 loss_ref, acc_ref):
    i = pl.program_id(0)
    n = pl.num_programs(0)
    logits = jnp.dot(x_ref[...], wg_ref[...], preferred_element_type=jnp.float32)
    lane = jax.lax.broadcasted_iota(jnp.int32, logits.shape, 1)
    logits = jnp.where(lane < E, logits, -jnp.inf)
    m = jnp.max(logits, axis=1, keepdims=True)
    eid = jnp.min(jnp.where(logits == m, lane, jnp.int32(2**30)), axis=1,
                  keepdims=True)
    eid_ref[...] = eid
    onehot = jnp.where((lane == eid) & (lane < E), jnp.float32(1.0),
                       jnp.float32(0.0))
    ra = jax.lax.broadcasted_iota(jnp.int32, (TPW, TPW), 0)
    rb = jax.lax.broadcasted_iota(jnp.int32, (TPW, TPW), 1)
    tri = jnp.where(rb < ra, jnp.float32(1.0), jnp.float32(0.0))
    for j in range(BTR // TPW):
        sub = onehot[j * TPW:(j + 1) * TPW, :]
        blk_ref[0, j, :] = jnp.sum(sub, axis=0)
        ranks = jnp.dot(tri, sub, preferred_element_type=jnp.float32)
        rnk_ref[j * TPW:(j + 1) * TPW, :] = jnp.sum(
            ranks * sub, axis=1, keepdims=True).astype(jnp.int32)

    @pl.when(i == 0)
    def _():
        acc_ref[...] = jnp.zeros_like(acc_ref)

    acc_ref[...] += jnp.sum(onehot, axis=0, keepdims=True)

    @pl.when(i == n - 1)
    def _():
        c = acc_ref[...]
        counts_ref[...] = c
        lane1 = lane[:1, :]
        mean = jnp.sum(jnp.where(lane1 < E, c, 0.0)) / jnp.float32(E)
        dev = jnp.where(lane1 < E, c - mean, 0.0)
        var = jnp.sum(dev * dev) / jnp.float32(E - 1)
        loss_ref[...] = jnp.full((1, 1), var / (mean * mean + 1e-10),
                                 jnp.float32)


def _router(x, wg_pad):
    nblk = BTR // TPW
    return pl.pallas_call(
        _router_body,
        grid=(T // BTR,),
        in_specs=[
            pl.BlockSpec((BTR, D), lambda i: (i, 0)),
            pl.BlockSpec((D, 128), lambda i: (0, 0)),
        ],
        out_specs=[
            pl.BlockSpec((BTR, 1), lambda i: (i, 0)),
            pl.BlockSpec((BTR, 1), lambda i: (i, 0)),
            pl.BlockSpec((1, 128), lambda i: (0, 0)),
            pl.BlockSpec((1, nblk, 128), lambda i: (i, 0, 0)),
            pl.BlockSpec((1, 1), lambda i: (0, 0)),
        ],
        out_shape=[
            jax.ShapeDtypeStruct((T, 1), jnp.int32),
            jax.ShapeDtypeStruct((T, 1), jnp.int32),
            jax.ShapeDtypeStruct((1, 128), jnp.float32),
            jax.ShapeDtypeStruct((T // BTR, nblk, 128), jnp.float32),
            jax.ShapeDtypeStruct((1, 1), jnp.float32),
        ],
        scratch_shapes=[pltpu.VMEM((1, 128), jnp.float32)],
        interpret=_INTERPRET,
    )(x, wg_pad)


def _sc_dispatch_body(eid_hbm, rnk_hbm, bases_hbm, x_hbm, dest_hbm, xs_hbm,
                      eid_v, rnk_v, dest_v, base_v, xrows_v, sem):
    wid = lax.axis_index("s") * 2 + lax.axis_index("c")
    tok0 = wid * TPW
    pltpu.sync_copy(eid_hbm.at[pl.ds(tok0, TPW)], eid_v)
    pltpu.sync_copy(rnk_hbm.at[pl.ds(tok0, TPW)], rnk_v)
    pltpu.sync_copy(bases_hbm.at[wid], base_v)
    for ch in range(TPW // 16):
        v = eid_v[pl.ds(ch * 16, 16)]
        b = plsc.load_gather(base_v, [v])
        dest_v[pl.ds(ch * 16, 16)] = b + rnk_v[pl.ds(ch * 16, 16)]
    pltpu.sync_copy(dest_v, dest_hbm.at[pl.ds(tok0, TPW)])
    pltpu.sync_copy(x_hbm.at[pl.ds(tok0, TPW)], xrows_v)
    pltpu.async_copy(xrows_v, xs_hbm.at[dest_v], sem).wait()


def _sc_dispatch(eid, rnk, bases, x):
    mesh = plsc.VectorSubcoreMesh(core_axis_name="c", subcore_axis_name="s")
    return pl.kernel(
        _sc_dispatch_body,
        out_type=[
            jax.ShapeDtypeStruct((T,), jnp.int32),
            jax.ShapeDtypeStruct((T, D), jnp.float32),
        ],
        mesh=mesh,
        scratch_types=[
            pltpu.VMEM((TPW,), jnp.int32),
            pltpu.VMEM((TPW,), jnp.int32),
            pltpu.VMEM((TPW,), jnp.int32),
            pltpu.VMEM((16,), jnp.int32),
            pltpu.VMEM((TPW, D), jnp.float32),
            pltpu.SemaphoreType.DMA,
        ],
        compiler_params=pltpu.CompilerParams(needs_layout_passes=False),
        interpret=_INTERPRET,
    )(eid, rnk, bases, x)


def _sc_combine_body(dest_hbm, y_hbm, out_hbm, dest_v, yrows_v, sem):
    wid = lax.axis_index("s") * 2 + lax.axis_index("c")
    tok0 = wid * TPW
    pltpu.sync_copy(dest_hbm.at[pl.ds(tok0, TPW)], dest_v)
    pltpu.async_copy(y_hbm.at[dest_v], yrows_v, sem).wait()
    pltpu.sync_copy(yrows_v, out_hbm.at[pl.ds(tok0, TPW)])


def _sc_combine(dest, y_sorted):
    mesh = plsc.VectorSubcoreMesh(core_axis_name="c", subcore_axis_name="s")
    return pl.kernel(
        _sc_combine_body,
        out_type=jax.ShapeDtypeStruct((T, D), jnp.float32),
        mesh=mesh,
        scratch_types=[
            pltpu.VMEM((TPW,), jnp.int32),
            pltpu.VMEM((TPW, D), jnp.float32),
            pltpu.SemaphoreType.DMA,
        ],
        compiler_params=pltpu.CompilerParams(needs_layout_passes=False),
        interpret=_INTERPRET,
    )(dest, y_sorted)


def _gmm_body(tile_ref, eidw_ref, offs_ref,
              x_ref, w1_ref, b1_ref, w2_ref, b2_ref, y_ref):
    w = pl.program_id(0)
    t = tile_ref[w]
    e = eidw_ref[w]
    s = offs_ref[e]
    epos = offs_ref[e + 1]
    rows = t * BT + jax.lax.broadcasted_iota(jnp.int32, (BT, 1), 0)
    mask = (rows >= s) & (rows < epos)
    h = jnp.maximum(
        jnp.dot(x_ref[...], w1_ref[0], preferred_element_type=jnp.float32)
        + b1_ref[0], 0.0)
    y = jnp.dot(h, w2_ref[0], preferred_element_type=jnp.float32) + b2_ref[0]
    y_ref[...] = jnp.where(mask, y, y_ref[...])


def _gmm(wu_tile, wu_eid, offs, x_sorted, W1, b1, W2, b2):
    grid_spec = pltpu.PrefetchScalarGridSpec(
        num_scalar_prefetch=3,
        grid=(NW,),
        in_specs=[
            pl.BlockSpec((BT, D), lambda w, tr, er, ofr: (tr[w], 0)),
            pl.BlockSpec((1, D, H), lambda w, tr, er, ofr: (er[w], 0, 0)),
            pl.BlockSpec((1, 1, H), lambda w, tr, er, ofr: (er[w], 0, 0)),
            pl.BlockSpec((1, H, D), lambda w, tr, er, ofr: (er[w], 0, 0)),
            pl.BlockSpec((1, 1, D), lambda w, tr, er, ofr: (er[w], 0, 0)),
        ],
        out_specs=pl.BlockSpec((BT, D), lambda w, tr, er, ofr: (tr[w], 0)),
    )
    return pl.pallas_call(
        _gmm_body,
        grid_spec=grid_spec,
        out_shape=jax.ShapeDtypeStruct((T, D), jnp.float32),
        compiler_params=pltpu.CompilerParams(
            dimension_semantics=("arbitrary",)),
        interpret=_INTERPRET,
    )(wu_tile, wu_eid, offs, x_sorted, W1,
      b1.reshape(E, 1, H), W2, b2.reshape(E, 1, D))


def _plan_work_units(offs9):
    interior = offs9[1:E]
    tstart = jnp.arange(NT, dtype=jnp.int32) * BT
    e_start = jnp.sum(interior[None, :] <= tstart[:, None], axis=1)
    e_end = jnp.sum(interior[None, :] <= (tstart + BT - 1)[:, None], axis=1)
    nw = e_end - e_start + 1
    starts = jnp.concatenate(
        [jnp.zeros((1,), jnp.int32), jnp.cumsum(nw)]).astype(jnp.int32)
    n = starts[NT]
    w = jnp.arange(NW, dtype=jnp.int32)
    wc = jnp.minimum(w, n - 1)
    t = jnp.sum(starts[None, :NT] <= wc[:, None], axis=1).astype(jnp.int32) - 1
    eid = e_start[t] + (wc - starts[t])
    return t.astype(jnp.int32), eid.astype(jnp.int32)


def kernel(x, Wg, W1, b1, W2, b2):
    wg_pad = jnp.zeros((D, 128), jnp.float32).at[:, :E].set(Wg)
    eid2d, rnk2d, counts128, blkcnt, loss11 = _router(x, wg_pad)
    eid = eid2d[:, 0]
    rnk = rnk2d[:, 0]
    counts = counts128[0, :E].astype(jnp.int32)
    loss = loss11[0, 0]
    offs9 = jnp.concatenate(
        [jnp.zeros((1,), jnp.int32), jnp.cumsum(counts)]).astype(jnp.int32)
    offs16 = jnp.zeros((16,), jnp.int32).at[:E + 1].set(offs9)

    blk = blkcnt.reshape(NWORK, 128)[:, :E].astype(jnp.int32)
    prefix = jnp.concatenate(
        [jnp.zeros((1, E), jnp.int32), jnp.cumsum(blk, axis=0)[:-1]], axis=0)
    bases = jnp.zeros((NWORK, 16), jnp.int32).at[:, :E].set(
        offs9[None, :E] + prefix)

    dest, x_sorted = _sc_dispatch(eid, rnk, bases, x)

    wu_tile, wu_eid = _plan_work_units(offs9)
    y_sorted = _gmm(wu_tile, wu_eid, offs16, x_sorted, W1, b1, W2, b2)

    out = _sc_combine(dest, y_sorted)
    return out, loss, loss


# speedup vs baseline: 1.2781x; 1.0787x over previous
import functools

import jax
import jax.numpy as jnp
from jax import lax
from jax.experimental import pallas as pl
from jax.experimental.pallas import tpu as pltpu
from jax.experimental.pallas import tpu_sc as plsc

_INTERPRET = False

E = 8
D = 768
H = 768
T = 4096
BTR = 512
BT = 256
NT = T // BT
NW = NT + E

NWORK = 32
TPW = T // NWORK


def _router_body(x_ref, wg_ref, eid_ref, rnk_ref, counts_ref, blk_ref,
                 loss_ref, acc_ref):
    i = pl.program_id(0)
    n = pl.num_programs(0)
    logits = jnp.dot(x_ref[...], wg_ref[...], preferred_element_type=jnp.float32)
    lane = jax.lax.broadcasted_iota(jnp.int32, logits.shape, 1)
    logits = jnp.where(lane < E, logits, -jnp.inf)
    m = jnp.max(logits, axis=1, keepdims=True)
    eid = jnp.min(jnp.where(logits == m, lane, jnp.int32(2**30)), axis=1,
                  keepdims=True)
    eid_ref[...] = eid
    onehot = jnp.where((lane == eid) & (lane < E), jnp.float32(1.0),
                       jnp.float32(0.0))
    ra = jax.lax.broadcasted_iota(jnp.int32, (TPW, TPW), 0)
    rb = jax.lax.broadcasted_iota(jnp.int32, (TPW, TPW), 1)
    tri = jnp.where(rb < ra, jnp.float32(1.0), jnp.float32(0.0))
    for j in range(BTR // TPW):
        sub = onehot[j * TPW:(j + 1) * TPW, :]
        blk_ref[0, j, :] = jnp.sum(sub, axis=0)
        ranks = jnp.dot(tri, sub, preferred_element_type=jnp.float32)
        rnk_ref[j * TPW:(j + 1) * TPW, :] = jnp.sum(
            ranks * sub, axis=1, keepdims=True).astype(jnp.int32)

    @pl.when(i == 0)
    def _():
        acc_ref[...] = jnp.zeros_like(acc_ref)

    acc_ref[...] += jnp.sum(onehot, axis=0, keepdims=True)

    @pl.when(i == n - 1)
    def _():
        c = acc_ref[...]
        counts_ref[...] = c
        lane1 = lane[:1, :]
        mean = jnp.sum(jnp.where(lane1 < E, c, 0.0)) / jnp.float32(E)
        dev = jnp.where(lane1 < E, c - mean, 0.0)
        var = jnp.sum(dev * dev) / jnp.float32(E - 1)
        loss_ref[...] = jnp.full((1, 1), var / (mean * mean + 1e-10),
                                 jnp.float32)


def _router(x, wg_pad):
    nblk = BTR // TPW
    return pl.pallas_call(
        _router_body,
        grid=(T // BTR,),
        in_specs=[
            pl.BlockSpec((BTR, D), lambda i: (i, 0)),
            pl.BlockSpec((D, 128), lambda i: (0, 0)),
        ],
        out_specs=[
            pl.BlockSpec((BTR, 1), lambda i: (i, 0)),
            pl.BlockSpec((BTR, 1), lambda i: (i, 0)),
            pl.BlockSpec((1, 128), lambda i: (0, 0)),
            pl.BlockSpec((1, nblk, 128), lambda i: (i, 0, 0)),
            pl.BlockSpec((1, 1), lambda i: (0, 0)),
        ],
        out_shape=[
            jax.ShapeDtypeStruct((T, 1), jnp.int32),
            jax.ShapeDtypeStruct((T, 1), jnp.int32),
            jax.ShapeDtypeStruct((1, 128), jnp.float32),
            jax.ShapeDtypeStruct((T // BTR, nblk, 128), jnp.float32),
            jax.ShapeDtypeStruct((1, 1), jnp.float32),
        ],
        scratch_shapes=[pltpu.VMEM((1, 128), jnp.float32)],
        interpret=_INTERPRET,
    )(x, wg_pad)


def _sc_dispatch_body(eid_hbm, rnk_hbm, bases_hbm, x_hbm, dest_hbm, xs_hbm,
                      eid_v, rnk_v, dest_v, base_v, xrows_v, sem):
    wid = lax.axis_index("s") * 2 + lax.axis_index("c")
    tok0 = wid * TPW
    pltpu.sync_copy(eid_hbm.at[pl.ds(tok0, TPW)], eid_v)
    pltpu.sync_copy(rnk_hbm.at[pl.ds(tok0, TPW)], rnk_v)
    pltpu.sync_copy(bases_hbm.at[wid], base_v)
    for ch in range(TPW // 16):
        v = eid_v[pl.ds(ch * 16, 16)]
        b = plsc.load_gather(base_v, [v])
        dest_v[pl.ds(ch * 16, 16)] = b + rnk_v[pl.ds(ch * 16, 16)]
    pltpu.sync_copy(dest_v, dest_hbm.at[pl.ds(tok0, TPW)])
    pltpu.sync_copy(x_hbm.at[pl.ds(tok0, TPW)], xrows_v)
    pltpu.async_copy(xrows_v, xs_hbm.at[dest_v], sem).wait()


def _sc_dispatch(eid, rnk, bases, x):
    mesh = plsc.VectorSubcoreMesh(core_axis_name="c", subcore_axis_name="s")
    return pl.kernel(
        _sc_dispatch_body,
        out_type=[
            jax.ShapeDtypeStruct((T,), jnp.int32),
            jax.ShapeDtypeStruct((T, D), jnp.float32),
        ],
        mesh=mesh,
        scratch_types=[
            pltpu.VMEM((TPW,), jnp.int32),
            pltpu.VMEM((TPW,), jnp.int32),
            pltpu.VMEM((TPW,), jnp.int32),
            pltpu.VMEM((16,), jnp.int32),
            pltpu.VMEM((TPW, D), jnp.float32),
            pltpu.SemaphoreType.DMA,
        ],
        compiler_params=pltpu.CompilerParams(needs_layout_passes=False),
        interpret=_INTERPRET,
    )(eid, rnk, bases, x)


def _sc_combine_body(dest_hbm, y_hbm, out_hbm, dest_v, yrows_v, sem):
    wid = lax.axis_index("s") * 2 + lax.axis_index("c")
    tok0 = wid * TPW
    pltpu.sync_copy(dest_hbm.at[pl.ds(tok0, TPW)], dest_v)
    pltpu.async_copy(y_hbm.at[dest_v], yrows_v, sem).wait()
    pltpu.sync_copy(yrows_v, out_hbm.at[pl.ds(tok0, TPW)])


def _sc_combine(dest, y_sorted):
    mesh = plsc.VectorSubcoreMesh(core_axis_name="c", subcore_axis_name="s")
    return pl.kernel(
        _sc_combine_body,
        out_type=jax.ShapeDtypeStruct((T, D), jnp.float32),
        mesh=mesh,
        scratch_types=[
            pltpu.VMEM((TPW,), jnp.int32),
            pltpu.VMEM((TPW, D), jnp.float32),
            pltpu.SemaphoreType.DMA,
        ],
        compiler_params=pltpu.CompilerParams(needs_layout_passes=False),
        interpret=_INTERPRET,
    )(dest, y_sorted)


def _gmm_body(tile_ref, eidw_ref, offs_ref,
              x_ref, w1_ref, b1_ref, w2_ref, b2_ref, y_ref):
    w = pl.program_id(0)
    t = tile_ref[w]
    e = eidw_ref[w]
    s = offs_ref[e]
    epos = offs_ref[e + 1]
    rows = t * BT + jax.lax.broadcasted_iota(jnp.int32, (BT, 1), 0)
    mask = (rows >= s) & (rows < epos)
    h = jnp.maximum(
        jnp.dot(x_ref[...], w1_ref[0].astype(jnp.bfloat16),
                preferred_element_type=jnp.float32) + b1_ref[0], 0.0)
    y = jnp.dot(h.astype(jnp.bfloat16), w2_ref[0].astype(jnp.bfloat16),
                preferred_element_type=jnp.float32) + b2_ref[0]
    y_ref[...] = jnp.where(mask, y, y_ref[...])


def _gmm(wu_tile, wu_eid, offs, x_sorted, W1, b1, W2, b2):
    grid_spec = pltpu.PrefetchScalarGridSpec(
        num_scalar_prefetch=3,
        grid=(NW,),
        in_specs=[
            pl.BlockSpec((BT, D), lambda w, tr, er, ofr: (tr[w], 0)),
            pl.BlockSpec((1, D, H), lambda w, tr, er, ofr: (er[w], 0, 0)),
            pl.BlockSpec((1, 1, H), lambda w, tr, er, ofr: (er[w], 0, 0)),
            pl.BlockSpec((1, H, D), lambda w, tr, er, ofr: (er[w], 0, 0)),
            pl.BlockSpec((1, 1, D), lambda w, tr, er, ofr: (er[w], 0, 0)),
        ],
        out_specs=pl.BlockSpec((BT, D), lambda w, tr, er, ofr: (tr[w], 0)),
    )
    return pl.pallas_call(
        _gmm_body,
        grid_spec=grid_spec,
        out_shape=jax.ShapeDtypeStruct((T, D), jnp.float32),
        compiler_params=pltpu.CompilerParams(
            dimension_semantics=("arbitrary",)),
        interpret=_INTERPRET,
    )(wu_tile, wu_eid, offs, x_sorted, W1,
      b1.reshape(E, 1, H), W2, b2.reshape(E, 1, D))


def _plan_work_units(offs9):
    interior = offs9[1:E]
    tstart = jnp.arange(NT, dtype=jnp.int32) * BT
    e_start = jnp.sum(interior[None, :] <= tstart[:, None], axis=1)
    e_end = jnp.sum(interior[None, :] <= (tstart + BT - 1)[:, None], axis=1)
    nw = e_end - e_start + 1
    starts = jnp.concatenate(
        [jnp.zeros((1,), jnp.int32), jnp.cumsum(nw)]).astype(jnp.int32)
    n = starts[NT]
    w = jnp.arange(NW, dtype=jnp.int32)
    wc = jnp.minimum(w, n - 1)
    t = jnp.sum(starts[None, :NT] <= wc[:, None], axis=1).astype(jnp.int32) - 1
    eid = e_start[t] + (wc - starts[t])
    return t.astype(jnp.int32), eid.astype(jnp.int32)


def kernel(x, Wg, W1, b1, W2, b2):
    wg_pad = jnp.zeros((D, 128), jnp.float32).at[:, :E].set(Wg)
    eid2d, rnk2d, counts128, blkcnt, loss11 = _router(x, wg_pad)
    eid = eid2d[:, 0]
    rnk = rnk2d[:, 0]
    counts = counts128[0, :E].astype(jnp.int32)
    loss = loss11[0, 0]
    offs9 = jnp.concatenate(
        [jnp.zeros((1,), jnp.int32), jnp.cumsum(counts)]).astype(jnp.int32)
    offs16 = jnp.zeros((16,), jnp.int32).at[:E + 1].set(offs9)

    blk = blkcnt.reshape(NWORK, 128)[:, :E].astype(jnp.int32)
    prefix = jnp.concatenate(
        [jnp.zeros((1, E), jnp.int32), jnp.cumsum(blk, axis=0)[:-1]], axis=0)
    bases = jnp.zeros((NWORK, 16), jnp.int32).at[:, :E].set(
        offs9[None, :E] + prefix)

    dest, x_sorted = _sc_dispatch(eid, rnk, bases, x)

    wu_tile, wu_eid = _plan_work_units(offs9)
    y_sorted = _gmm(wu_tile, wu_eid, offs16, x_sorted, W1, b1, W2, b2)

    out = _sc_combine(dest, y_sorted)
    return out, loss, loss


# speedup vs baseline: 1.2998x; 1.0170x over previous
import functools

import jax
import jax.numpy as jnp
from jax import lax
from jax.experimental import pallas as pl
from jax.experimental.pallas import tpu as pltpu
from jax.experimental.pallas import tpu_sc as plsc

_INTERPRET = False

E = 8
D = 768
H = 768
T = 4096
BTR = 512
BT = 512
NT = T // BT
NW = NT + E

NWORK = 32
TPW = T // NWORK


def _router_body(x_ref, wg_ref, eid_ref, rnk_ref, counts_ref, blk_ref,
                 loss_ref, acc_ref):
    i = pl.program_id(0)
    n = pl.num_programs(0)
    logits = jnp.dot(x_ref[...], wg_ref[...], preferred_element_type=jnp.float32)
    lane = jax.lax.broadcasted_iota(jnp.int32, logits.shape, 1)
    logits = jnp.where(lane < E, logits, -jnp.inf)
    m = jnp.max(logits, axis=1, keepdims=True)
    eid = jnp.min(jnp.where(logits == m, lane, jnp.int32(2**30)), axis=1,
                  keepdims=True)
    eid_ref[...] = eid
    onehot = jnp.where((lane == eid) & (lane < E), jnp.float32(1.0),
                       jnp.float32(0.0))
    ra = jax.lax.broadcasted_iota(jnp.int32, (TPW, TPW), 0)
    rb = jax.lax.broadcasted_iota(jnp.int32, (TPW, TPW), 1)
    tri = jnp.where(rb < ra, jnp.float32(1.0), jnp.float32(0.0))
    for j in range(BTR // TPW):
        sub = onehot[j * TPW:(j + 1) * TPW, :]
        blk_ref[0, j, :] = jnp.sum(sub, axis=0)
        ranks = jnp.dot(tri, sub, preferred_element_type=jnp.float32)
        rnk_ref[j * TPW:(j + 1) * TPW, :] = jnp.sum(
            ranks * sub, axis=1, keepdims=True).astype(jnp.int32)

    @pl.when(i == 0)
    def _():
        acc_ref[...] = jnp.zeros_like(acc_ref)

    acc_ref[...] += jnp.sum(onehot, axis=0, keepdims=True)

    @pl.when(i == n - 1)
    def _():
        c = acc_ref[...]
        counts_ref[...] = c
        lane1 = lane[:1, :]
        mean = jnp.sum(jnp.where(lane1 < E, c, 0.0)) / jnp.float32(E)
        dev = jnp.where(lane1 < E, c - mean, 0.0)
        var = jnp.sum(dev * dev) / jnp.float32(E - 1)
        loss_ref[...] = jnp.full((1, 1), var / (mean * mean + 1e-10),
                                 jnp.float32)


def _router(x, wg_pad):
    nblk = BTR // TPW
    return pl.pallas_call(
        _router_body,
        grid=(T // BTR,),
        in_specs=[
            pl.BlockSpec((BTR, D), lambda i: (i, 0)),
            pl.BlockSpec((D, 128), lambda i: (0, 0)),
        ],
        out_specs=[
            pl.BlockSpec((BTR, 1), lambda i: (i, 0)),
            pl.BlockSpec((BTR, 1), lambda i: (i, 0)),
            pl.BlockSpec((1, 128), lambda i: (0, 0)),
            pl.BlockSpec((1, nblk, 128), lambda i: (i, 0, 0)),
            pl.BlockSpec((1, 1), lambda i: (0, 0)),
        ],
        out_shape=[
            jax.ShapeDtypeStruct((T, 1), jnp.int32),
            jax.ShapeDtypeStruct((T, 1), jnp.int32),
            jax.ShapeDtypeStruct((1, 128), jnp.float32),
            jax.ShapeDtypeStruct((T // BTR, nblk, 128), jnp.float32),
            jax.ShapeDtypeStruct((1, 1), jnp.float32),
        ],
        scratch_shapes=[pltpu.VMEM((1, 128), jnp.float32)],
        interpret=_INTERPRET,
    )(x, wg_pad)


def _sc_dispatch_body(eid_hbm, rnk_hbm, bases_hbm, x_hbm, dest_hbm, xs_hbm,
                      eid_v, rnk_v, dest_v, base_v, xrows_v, sem):
    wid = lax.axis_index("s") * 2 + lax.axis_index("c")
    tok0 = wid * TPW
    pltpu.sync_copy(eid_hbm.at[pl.ds(tok0, TPW)], eid_v)
    pltpu.sync_copy(rnk_hbm.at[pl.ds(tok0, TPW)], rnk_v)
    pltpu.sync_copy(bases_hbm.at[wid], base_v)
    for ch in range(TPW // 16):
        v = eid_v[pl.ds(ch * 16, 16)]
        b = plsc.load_gather(base_v, [v])
        dest_v[pl.ds(ch * 16, 16)] = b + rnk_v[pl.ds(ch * 16, 16)]
    pltpu.sync_copy(dest_v, dest_hbm.at[pl.ds(tok0, TPW)])
    pltpu.sync_copy(x_hbm.at[pl.ds(tok0, TPW)], xrows_v)
    pltpu.async_copy(xrows_v, xs_hbm.at[dest_v], sem).wait()


def _sc_dispatch(eid, rnk, bases, x):
    mesh = plsc.VectorSubcoreMesh(core_axis_name="c", subcore_axis_name="s")
    return pl.kernel(
        _sc_dispatch_body,
        out_type=[
            jax.ShapeDtypeStruct((T,), jnp.int32),
            jax.ShapeDtypeStruct((T, D), jnp.float32),
        ],
        mesh=mesh,
        scratch_types=[
            pltpu.VMEM((TPW,), jnp.int32),
            pltpu.VMEM((TPW,), jnp.int32),
            pltpu.VMEM((TPW,), jnp.int32),
            pltpu.VMEM((16,), jnp.int32),
            pltpu.VMEM((TPW, D), jnp.float32),
            pltpu.SemaphoreType.DMA,
        ],
        compiler_params=pltpu.CompilerParams(needs_layout_passes=False),
        interpret=_INTERPRET,
    )(eid, rnk, bases, x)


def _sc_combine_body(dest_hbm, y_hbm, out_hbm, dest_v, yrows_v, sem):
    wid = lax.axis_index("s") * 2 + lax.axis_index("c")
    tok0 = wid * TPW
    pltpu.sync_copy(dest_hbm.at[pl.ds(tok0, TPW)], dest_v)
    pltpu.async_copy(y_hbm.at[dest_v], yrows_v, sem).wait()
    pltpu.sync_copy(yrows_v, out_hbm.at[pl.ds(tok0, TPW)])


def _sc_combine(dest, y_sorted):
    mesh = plsc.VectorSubcoreMesh(core_axis_name="c", subcore_axis_name="s")
    return pl.kernel(
        _sc_combine_body,
        out_type=jax.ShapeDtypeStruct((T, D), jnp.float32),
        mesh=mesh,
        scratch_types=[
            pltpu.VMEM((TPW,), jnp.int32),
            pltpu.VMEM((TPW, D), jnp.float32),
            pltpu.SemaphoreType.DMA,
        ],
        compiler_params=pltpu.CompilerParams(needs_layout_passes=False),
        interpret=_INTERPRET,
    )(dest, y_sorted)


def _gmm_body(tile_ref, eidw_ref, offs_ref,
              x_ref, w1_ref, b1_ref, w2_ref, b2_ref, y_ref):
    w = pl.program_id(0)
    t = tile_ref[w]
    e = eidw_ref[w]
    s = offs_ref[e]
    epos = offs_ref[e + 1]
    rows = t * BT + jax.lax.broadcasted_iota(jnp.int32, (BT, 1), 0)
    mask = (rows >= s) & (rows < epos)
    h = jnp.maximum(
        jnp.dot(x_ref[...], w1_ref[0].astype(jnp.bfloat16),
                preferred_element_type=jnp.float32) + b1_ref[0], 0.0)
    y = jnp.dot(h.astype(jnp.bfloat16), w2_ref[0].astype(jnp.bfloat16),
                preferred_element_type=jnp.float32) + b2_ref[0]
    y_ref[...] = jnp.where(mask, y, y_ref[...])


def _gmm(wu_tile, wu_eid, offs, x_sorted, W1, b1, W2, b2):
    grid_spec = pltpu.PrefetchScalarGridSpec(
        num_scalar_prefetch=3,
        grid=(NW,),
        in_specs=[
            pl.BlockSpec((BT, D), lambda w, tr, er, ofr: (tr[w], 0)),
            pl.BlockSpec((1, D, H), lambda w, tr, er, ofr: (er[w], 0, 0)),
            pl.BlockSpec((1, 1, H), lambda w, tr, er, ofr: (er[w], 0, 0)),
            pl.BlockSpec((1, H, D), lambda w, tr, er, ofr: (er[w], 0, 0)),
            pl.BlockSpec((1, 1, D), lambda w, tr, er, ofr: (er[w], 0, 0)),
        ],
        out_specs=pl.BlockSpec((BT, D), lambda w, tr, er, ofr: (tr[w], 0)),
    )
    return pl.pallas_call(
        _gmm_body,
        grid_spec=grid_spec,
        out_shape=jax.ShapeDtypeStruct((T, D), jnp.float32),
        compiler_params=pltpu.CompilerParams(
            dimension_semantics=("arbitrary",)),
        interpret=_INTERPRET,
    )(wu_tile, wu_eid, offs, x_sorted, W1,
      b1.reshape(E, 1, H), W2, b2.reshape(E, 1, D))


def _plan_work_units(offs9):
    interior = offs9[1:E]
    tstart = jnp.arange(NT, dtype=jnp.int32) * BT
    e_start = jnp.sum(interior[None, :] <= tstart[:, None], axis=1)
    e_end = jnp.sum(interior[None, :] <= (tstart + BT - 1)[:, None], axis=1)
    nw = e_end - e_start + 1
    starts = jnp.concatenate(
        [jnp.zeros((1,), jnp.int32), jnp.cumsum(nw)]).astype(jnp.int32)
    n = starts[NT]
    w = jnp.arange(NW, dtype=jnp.int32)
    wc = jnp.minimum(w, n - 1)
    t = jnp.sum(starts[None, :NT] <= wc[:, None], axis=1).astype(jnp.int32) - 1
    eid = e_start[t] + (wc - starts[t])
    return t.astype(jnp.int32), eid.astype(jnp.int32)


def kernel(x, Wg, W1, b1, W2, b2):
    wg_pad = jnp.zeros((D, 128), jnp.float32).at[:, :E].set(Wg)
    eid2d, rnk2d, counts128, blkcnt, loss11 = _router(x, wg_pad)
    eid = eid2d[:, 0]
    rnk = rnk2d[:, 0]
    counts = counts128[0, :E].astype(jnp.int32)
    loss = loss11[0, 0]
    offs9 = jnp.concatenate(
        [jnp.zeros((1,), jnp.int32), jnp.cumsum(counts)]).astype(jnp.int32)
    offs16 = jnp.zeros((16,), jnp.int32).at[:E + 1].set(offs9)

    blk = blkcnt.reshape(NWORK, 128)[:, :E].astype(jnp.int32)
    prefix = jnp.concatenate(
        [jnp.zeros((1, E), jnp.int32), jnp.cumsum(blk, axis=0)[:-1]], axis=0)
    bases = jnp.zeros((NWORK, 16), jnp.int32).at[:, :E].set(
        offs9[None, :E] + prefix)

    dest, x_sorted = _sc_dispatch(eid, rnk, bases, x)

    wu_tile, wu_eid = _plan_work_units(offs9)
    y_sorted = _gmm(wu_tile, wu_eid, offs16, x_sorted, W1, b1, W2, b2)

    out = _sc_combine(dest, y_sorted)
    return out, loss, loss


# speedup vs baseline: 1.3564x; 1.0435x over previous
import jax
import jax.numpy as jnp
from jax import lax
from jax.experimental import pallas as pl
from jax.experimental.pallas import tpu as pltpu
from jax.experimental.pallas import tpu_sc as plsc

_INTERPRET = False

E = 8
D = 768
H = 768
T = 4096
BTR = 512
BT = 512
NT = T // BT
NW = NT + E

NWORK = 32
TPW = T // NWORK


def _router_body(x_ref, wg_ref, eid_ref, rnk_ref, bases_ref, meta_ref,
                 loss_ref, acc_ref, blk_ref):
    i = pl.program_id(0)
    n = pl.num_programs(0)
    nblk = BTR // TPW
    logits = jnp.dot(x_ref[...], wg_ref[...], preferred_element_type=jnp.float32)
    lane = jax.lax.broadcasted_iota(jnp.int32, logits.shape, 1)
    logits = jnp.where(lane < E, logits, -jnp.inf)
    m = jnp.max(logits, axis=1, keepdims=True)
    eid = jnp.min(jnp.where(logits == m, lane, jnp.int32(2**30)), axis=1,
                  keepdims=True)
    eid_ref[...] = eid
    onehot = jnp.where((lane == eid) & (lane < E), jnp.float32(1.0),
                       jnp.float32(0.0))
    ra = jax.lax.broadcasted_iota(jnp.int32, (TPW, TPW), 0)
    rb = jax.lax.broadcasted_iota(jnp.int32, (TPW, TPW), 1)
    tri = jnp.where(rb < ra, jnp.float32(1.0), jnp.float32(0.0))
    for j in range(nblk):
        sub = onehot[j * TPW:(j + 1) * TPW, :]
        blk_ref[pl.ds(i * nblk + j, 1), :] = jnp.sum(sub, axis=0,
                                                     keepdims=True)
        ranks = jnp.dot(tri, sub, preferred_element_type=jnp.float32)
        rnk_ref[j * TPW:(j + 1) * TPW, :] = jnp.sum(
            ranks * sub, axis=1, keepdims=True).astype(jnp.int32)

    @pl.when(i == 0)
    def _():
        acc_ref[...] = jnp.zeros_like(acc_ref)

    acc_ref[...] += jnp.sum(onehot, axis=0, keepdims=True)

    @pl.when(i == n - 1)
    def _():
        c = acc_ref[...]
        lane1 = lane[:1, :]
        mean = jnp.sum(jnp.where(lane1 < E, c, 0.0)) / jnp.float32(E)
        dev = jnp.where(lane1 < E, c - mean, 0.0)
        var = jnp.sum(dev * dev) / jnp.float32(E - 1)
        loss_ref[...] = jnp.full((1, 1), var / (mean * mean + 1e-10),
                                 jnp.float32)
        ua = jax.lax.broadcasted_iota(jnp.int32, (128, 128), 0)
        ub = jax.lax.broadcasted_iota(jnp.int32, (128, 128), 1)
        triu = jnp.where(ua < ub, jnp.float32(1.0), jnp.float32(0.0))
        offs_row = jnp.dot(c, triu, preferred_element_type=jnp.float32)
        wa = jax.lax.broadcasted_iota(jnp.int32, (NWORK, NWORK), 0)
        wb = jax.lax.broadcasted_iota(jnp.int32, (NWORK, NWORK), 1)
        tri32 = jnp.where(wb < wa, jnp.float32(1.0), jnp.float32(0.0))
        pre = jnp.dot(tri32, blk_ref[...], preferred_element_type=jnp.float32)
        bases_ref[...] = (pre + offs_row).astype(jnp.int32)
        interior = (lane1 >= 1) & (lane1 < E)
        starts = [jnp.float32(0.0)]
        es_list, ee_list = [], []
        for t in range(NT):
            es = jnp.sum(jnp.where(interior & (offs_row <= t * BT), 1.0, 0.0))
            ee = jnp.sum(jnp.where(
                interior & (offs_row <= t * BT + (BT - 1)), 1.0, 0.0))
            es_list.append(es)
            ee_list.append(ee)
            starts.append(starts[-1] + (ee - es + 1.0))
        ntot = starts[NT]
        wlane = lane1.astype(jnp.float32)
        wc = jnp.minimum(wlane, ntot - 1.0)
        t_of_w = jnp.zeros_like(wlane)
        for t in range(NT):
            t_of_w += jnp.where(starts[t] <= wc, 1.0, 0.0)
        t_of_w -= 1.0
        e_of_w = jnp.zeros_like(wlane)
        for t in range(NT):
            sel = t_of_w == jnp.float32(t)
            e_of_w = jnp.where(sel, es_list[t] + (wc - starts[t]), e_of_w)
        meta_ref[0:1, :] = offs_row.astype(jnp.int32)
        meta_ref[1:2, :] = t_of_w.astype(jnp.int32)
        meta_ref[2:3, :] = e_of_w.astype(jnp.int32)
        for r in range(3, 8):
            meta_ref[r:r + 1, :] = jnp.zeros((1, 128), jnp.int32)


def _router(x, wg_pad):
    return pl.pallas_call(
        _router_body,
        grid=(T // BTR,),
        in_specs=[
            pl.BlockSpec((BTR, D), lambda i: (i, 0)),
            pl.BlockSpec((D, 128), lambda i: (0, 0)),
        ],
        out_specs=[
            pl.BlockSpec((BTR, 1), lambda i: (i, 0)),
            pl.BlockSpec((BTR, 1), lambda i: (i, 0)),
            pl.BlockSpec((NWORK, 128), lambda i: (0, 0)),
            pl.BlockSpec((8, 128), lambda i: (0, 0)),
            pl.BlockSpec((1, 1), lambda i: (0, 0)),
        ],
        out_shape=[
            jax.ShapeDtypeStruct((T, 1), jnp.int32),
            jax.ShapeDtypeStruct((T, 1), jnp.int32),
            jax.ShapeDtypeStruct((NWORK, 128), jnp.int32),
            jax.ShapeDtypeStruct((8, 128), jnp.int32),
            jax.ShapeDtypeStruct((1, 1), jnp.float32),
        ],
        scratch_shapes=[pltpu.VMEM((1, 128), jnp.float32),
                        pltpu.VMEM((NWORK, 128), jnp.float32)],
        interpret=_INTERPRET,
    )(x, wg_pad)


def _sc_dispatch_body(eid_hbm, rnk_hbm, bases_hbm, x_hbm, dest_hbm, xs_hbm,
                      eid_v, rnk_v, dest_v, base_v, xrows_v, sem):
    wid = lax.axis_index("s") * 2 + lax.axis_index("c")
    tok0 = wid * TPW
    pltpu.sync_copy(eid_hbm.at[pl.ds(tok0, TPW)], eid_v)
    pltpu.sync_copy(rnk_hbm.at[pl.ds(tok0, TPW)], rnk_v)
    pltpu.sync_copy(bases_hbm.at[wid], base_v)
    for ch in range(TPW // 16):
        v = eid_v[pl.ds(ch * 16, 16)]
        b = plsc.load_gather(base_v, [v])
        dest_v[pl.ds(ch * 16, 16)] = b + rnk_v[pl.ds(ch * 16, 16)]
    pltpu.sync_copy(dest_v, dest_hbm.at[pl.ds(tok0, TPW)])
    pltpu.sync_copy(x_hbm.at[pl.ds(tok0, TPW)], xrows_v)
    pltpu.async_copy(xrows_v, xs_hbm.at[dest_v], sem).wait()


def _sc_dispatch(eid, rnk, bases, x):
    mesh = plsc.VectorSubcoreMesh(core_axis_name="c", subcore_axis_name="s")
    return pl.kernel(
        _sc_dispatch_body,
        out_type=[
            jax.ShapeDtypeStruct((T,), jnp.int32),
            jax.ShapeDtypeStruct((T, D), jnp.float32),
        ],
        mesh=mesh,
        scratch_types=[
            pltpu.VMEM((TPW,), jnp.int32),
            pltpu.VMEM((TPW,), jnp.int32),
            pltpu.VMEM((TPW,), jnp.int32),
            pltpu.VMEM((128,), jnp.int32),
            pltpu.VMEM((TPW, D), jnp.float32),
            pltpu.SemaphoreType.DMA,
        ],
        compiler_params=pltpu.CompilerParams(needs_layout_passes=False),
        interpret=_INTERPRET,
    )(eid, rnk, bases, x)


def _sc_combine_body(dest_hbm, y_hbm, out_hbm, dest_v, yrows_v, sem):
    wid = lax.axis_index("s") * 2 + lax.axis_index("c")
    tok0 = wid * TPW
    pltpu.sync_copy(dest_hbm.at[pl.ds(tok0, TPW)], dest_v)
    pltpu.async_copy(y_hbm.at[dest_v], yrows_v, sem).wait()
    pltpu.sync_copy(yrows_v, out_hbm.at[pl.ds(tok0, TPW)])


def _sc_combine(dest, y_sorted):
    mesh = plsc.VectorSubcoreMesh(core_axis_name="c", subcore_axis_name="s")
    return pl.kernel(
        _sc_combine_body,
        out_type=jax.ShapeDtypeStruct((T, D), jnp.float32),
        mesh=mesh,
        scratch_types=[
            pltpu.VMEM((TPW,), jnp.int32),
            pltpu.VMEM((TPW, D), jnp.float32),
            pltpu.SemaphoreType.DMA,
        ],
        compiler_params=pltpu.CompilerParams(needs_layout_passes=False),
        interpret=_INTERPRET,
    )(dest, y_sorted)


def _gmm_body(tile_ref, eidw_ref, offs_ref,
              x_ref, w1_ref, b1_ref, w2_ref, b2_ref, y_ref):
    w = pl.program_id(0)
    t = tile_ref[w]
    e = eidw_ref[w]
    s = offs_ref[e]
    epos = offs_ref[e + 1]
    rows = t * BT + jax.lax.broadcasted_iota(jnp.int32, (BT, 1), 0)
    mask = (rows >= s) & (rows < epos)
    h = jnp.maximum(
        jnp.dot(x_ref[...], w1_ref[0].astype(jnp.bfloat16),
                preferred_element_type=jnp.float32) + b1_ref[0], 0.0)
    y = jnp.dot(h.astype(jnp.bfloat16), w2_ref[0].astype(jnp.bfloat16),
                preferred_element_type=jnp.float32) + b2_ref[0]
    y_ref[...] = jnp.where(mask, y, y_ref[...])


def _gmm(wu_tile, wu_eid, offs, x_sorted, W1, b1, W2, b2):
    grid_spec = pltpu.PrefetchScalarGridSpec(
        num_scalar_prefetch=3,
        grid=(NW,),
        in_specs=[
            pl.BlockSpec((BT, D), lambda w, tr, er, ofr: (tr[w], 0)),
            pl.BlockSpec((1, D, H), lambda w, tr, er, ofr: (er[w], 0, 0)),
            pl.BlockSpec((1, 1, H), lambda w, tr, er, ofr: (er[w], 0, 0)),
            pl.BlockSpec((1, H, D), lambda w, tr, er, ofr: (er[w], 0, 0)),
            pl.BlockSpec((1, 1, D), lambda w, tr, er, ofr: (er[w], 0, 0)),
        ],
        out_specs=pl.BlockSpec((BT, D), lambda w, tr, er, ofr: (tr[w], 0)),
    )
    return pl.pallas_call(
        _gmm_body,
        grid_spec=grid_spec,
        out_shape=jax.ShapeDtypeStruct((T, D), jnp.float32),
        compiler_params=pltpu.CompilerParams(
            dimension_semantics=("arbitrary",)),
        interpret=_INTERPRET,
    )(wu_tile, wu_eid, offs, x_sorted, W1,
      b1.reshape(E, 1, H), W2, b2.reshape(E, 1, D))


def kernel(x, Wg, W1, b1, W2, b2):
    wg_pad = jnp.zeros((D, 128), jnp.float32).at[:, :E].set(Wg)
    eid2d, rnk2d, bases, meta, loss11 = _router(x, wg_pad)
    eid = eid2d[:, 0]
    rnk = rnk2d[:, 0]
    loss = loss11[0, 0]
    offs16 = meta[0, :16]
    wu_tile = meta[1, :NW]
    wu_eid = meta[2, :NW]

    dest, x_sorted = _sc_dispatch(eid, rnk, bases, x)
    y_sorted = _gmm(wu_tile, wu_eid, offs16, x_sorted, W1, b1, W2, b2)
    out = _sc_combine(dest, y_sorted)
    return out, loss, loss
